# baseline (speedup 1.0000x reference)
import functools

import jax
import jax.numpy as jnp
from jax import lax
from jax.experimental import pallas as pl
from jax.experimental.pallas import tpu as pltpu
from jax.experimental.pallas import tpu_sc as plsc

D = 128
NC = 2
NS = 16
NW = NC * NS
L = 16
CHUNK = 96
NBUF = 2
N_PAD = 10240
HSIZE = 10224
ROWS_PER_TILE = N_PAD // NS
ZROWS = 64


def _sc_aggregate(x, packed, n_chunks):
    mesh = plsc.VectorSubcoreMesh(core_axis_name="c", subcore_axis_name="s")

    @functools.partial(
        pl.kernel,
        out_type=(
            jax.ShapeDtypeStruct((N_PAD, D), jnp.float32),
            jax.ShapeDtypeStruct((N_PAD, D), jnp.float32),
            jax.ShapeDtypeStruct((NW, HSIZE), jnp.float32),
        ),
        mesh=mesh,
        scratch_types=[
            pltpu.VMEM((n_chunks * CHUNK,), jnp.int32),
            pltpu.VMEM((NBUF, CHUNK), jnp.int32),
            pltpu.VMEM((NBUF, CHUNK), jnp.int32),
            pltpu.VMEM((NBUF, CHUNK, D), jnp.float32),
            pltpu.VMEM((HSIZE,), jnp.float32),
            pltpu.VMEM_SHARED((N_PAD, D), jnp.float32),
            pltpu.SemaphoreType.DMA((NBUF,)),
        ],
        compiler_params=pltpu.CompilerParams(needs_layout_passes=False),
    )
    def agg(x_hbm, packed_hbm, psum_a, psum_b, hist_hbm,
            slab_v, src_st, dst_st, rows2, hist_v, acc_sh, gsem):
        cid = lax.axis_index("c")
        sid = lax.axis_index("s")
        wid = cid * NS + sid
        rows_v = rows2.at[0]
        ones16 = jnp.ones((L,), jnp.float32)

        def unpack(c, b):
            def u(i, _):
                w = slab_v[pl.ds(c * CHUNK + i * L, L)]
                src_st[b, pl.ds(i * L, L)] = w & jnp.int32(0xFFFF)
                dst_st[b, pl.ds(i * L, L)] = lax.shift_right_logical(w, 16)
                return 0
            lax.fori_loop(0, CHUNK // L, u, 0)

        def zrow(i, _):
            def zcol(j, _):
                rows_v[i, pl.ds(j * L, L)] = jnp.zeros((L,), jnp.float32)
                return 0
            lax.fori_loop(0, D // L, zcol, 0)
            return 0
        lax.fori_loop(0, ZROWS, zrow, 0)

        def zhist(i, _):
            hist_v[pl.ds(i * L, L)] = jnp.zeros((L,), jnp.float32)
            return 0
        lax.fori_loop(0, HSIZE // L, zhist, 0)

        base = sid * ROWS_PER_TILE
        def zacc(t, _):
            pltpu.sync_copy(rows2.at[0, pl.ds(0, ZROWS)],
                            acc_sh.at[pl.ds(base + t * ZROWS, ZROWS)])
            return 0
        lax.fori_loop(0, ROWS_PER_TILE // ZROWS, zacc, 0)

        pltpu.sync_copy(packed_hbm.at[wid], slab_v)

        for b in range(NBUF):
            unpack(b, b)
            pltpu.async_copy(x_hbm.at[src_st.at[b]], rows2.at[b], gsem.at[b])

        plsc.subcore_barrier()

        scope = jax.named_scope("edge_loop")
        scope.__enter__()

        def pipe_body(t, _):
            j = t * NBUF
            for b in range(NBUF):
                cur = j + b
                def cnt(i, _):
                    idx16 = dst_st[b, pl.ds(i * L, L)]
                    plsc.addupdate_scatter(hist_v, [idx16], ones16)
                    return 0
                lax.fori_loop(0, CHUNK // L, cnt, 0)
                pltpu.make_async_copy(
                    x_hbm.at[src_st.at[b]], rows2.at[b], gsem.at[b]).wait()
                pltpu.sync_copy(rows2.at[b], acc_sh.at[dst_st.at[b]], add=True)
                nxt = cur + NBUF
                @pl.when(nxt < n_chunks)
                def _():
                    unpack(nxt, b)
                    pltpu.async_copy(
                        x_hbm.at[src_st.at[b]], rows2.at[b], gsem.at[b])
            return 0
        lax.fori_loop(0, n_chunks // NBUF, pipe_body, 0)
        scope.__exit__(None, None, None)

        pltpu.sync_copy(hist_v, hist_hbm.at[wid])
        plsc.subcore_barrier()

        sl = pl.ds(base, ROWS_PER_TILE)
        @pl.when(cid == 0)
        def _():
            pltpu.sync_copy(acc_sh.at[sl], psum_a.at[sl])
        @pl.when(cid == 1)
        def _():
            pltpu.sync_copy(acc_sh.at[sl], psum_b.at[sl])

    return agg(x, packed)


def _tc_self(x, W_self, b_self, b_neigh):
    n = x.shape[0]
    blk = 1024
    grid = (-(-n // blk),)

    def body(x_ref, ws_ref, bs_ref, bn_ref, o_ref):
        dn = (((1,), (1,)), ((), ()))
        o_ref[:] = (
            lax.dot_general(x_ref[:], ws_ref[:], dn,
                            preferred_element_type=jnp.float32)
            + bs_ref[:] + bn_ref[:]
        )

    return pl.pallas_call(
        body,
        grid=grid,
        in_specs=[
            pl.BlockSpec((blk, D), lambda i: (i, 0)),
            pl.BlockSpec((D, D), lambda i: (0, 0)),
            pl.BlockSpec((1, D), lambda i: (0, 0)),
            pl.BlockSpec((1, D), lambda i: (0, 0)),
        ],
        out_specs=pl.BlockSpec((blk, D), lambda i: (i, 0)),
        out_shape=jax.ShapeDtypeStruct((n, D), jnp.float32),
    )(x, W_self, b_self.reshape(1, D), b_neigh.reshape(1, D))


def _tc_combine(self_part, psum_a, psum_b, pcnt, W_neigh):
    n = self_part.shape[0]
    blk = 1024
    grid = (-(-n // blk),)

    def body(sp_ref, pa_ref, pb_ref, pc_ref, wn_ref, o_ref):
        s = pa_ref[:] + pb_ref[:]
        cnt = jnp.sum(pc_ref[:], axis=0)[:, None]
        mean = s / jnp.maximum(cnt, 1.0)
        dn = (((1,), (1,)), ((), ()))
        o_ref[:] = sp_ref[:] + lax.dot_general(
            mean, wn_ref[:], dn, preferred_element_type=jnp.float32)

    return pl.pallas_call(
        body,
        grid=grid,
        in_specs=[
            pl.BlockSpec((blk, D), lambda i: (i, 0)),
            pl.BlockSpec((blk, D), lambda i: (i, 0)),
            pl.BlockSpec((blk, D), lambda i: (i, 0)),
            pl.BlockSpec((NW, blk), lambda i: (0, i)),
            pl.BlockSpec((D, D), lambda i: (0, 0)),
        ],
        out_specs=pl.BlockSpec((blk, D), lambda i: (i, 0)),
        out_shape=jax.ShapeDtypeStruct((n, D), jnp.float32),
    )(self_part, psum_a, psum_b, pcnt, W_neigh)


def kernel(x, edge_index, W_self, b_self, W_neigh, b_neigh):
    n = x.shape[0]
    src = edge_index[0].astype(jnp.int32)
    dst = edge_index[1].astype(jnp.int32)
    e = src.shape[0]
    n_chunks = -(-e // (NW * CHUNK))
    n_chunks = max(-(-n_chunks // NBUF) * NBUF, NBUF)
    pad = NW * CHUNK * n_chunks - e
    pad_idx = jnp.arange(pad, dtype=jnp.int32)
    pad_dst = n + jnp.minimum(pad_idx & 127, HSIZE - n - 1)
    src_p = jnp.concatenate([src, jnp.zeros((pad,), jnp.int32)])
    dst_p = jnp.concatenate([dst, pad_dst])
    packed = (src_p | (dst_p << 16)).reshape(NW, n_chunks * CHUNK)

    psum_a, psum_b, pcnt = _sc_aggregate(x, packed, n_chunks)

    self_part = _tc_self(x, W_self, b_self, b_neigh)
    return _tc_combine(self_part, psum_a, psum_b, pcnt, W_neigh)


# speedup vs baseline: 2.6126x; 2.6126x over previous
import functools

import jax
import jax.numpy as jnp
from jax import lax
from jax.experimental import pallas as pl
from jax.experimental.pallas import tpu as pltpu
from jax.experimental.pallas import tpu_sc as plsc

D = 128
NC = 2
NS = 16
NW = NC * NS
L = 16
CHUNK = 96
NBUF = 2
N_PAD = 10240
HSIZE = 10224
ROWS_PER_TILE = N_PAD // NS
ZROWS = 64


def _sc_aggregate(x, packed, n_chunks):
    mesh = plsc.VectorSubcoreMesh(core_axis_name="c", subcore_axis_name="s")

    @functools.partial(
        pl.kernel,
        out_type=(
            jax.ShapeDtypeStruct((N_PAD, D), jnp.float32),
            jax.ShapeDtypeStruct((N_PAD, D), jnp.float32),
            jax.ShapeDtypeStruct((NW, HSIZE), jnp.float32),
        ),
        mesh=mesh,
        scratch_types=[
            pltpu.VMEM((n_chunks * CHUNK,), jnp.int32),
            pltpu.VMEM((NBUF, CHUNK), jnp.int32),
            pltpu.VMEM((NBUF, CHUNK), jnp.int32),
            pltpu.VMEM((NBUF, CHUNK, D), jnp.float32),
            pltpu.VMEM((HSIZE,), jnp.float32),
            pltpu.VMEM_SHARED((N_PAD, D), jnp.float32),
            pltpu.SemaphoreType.DMA((NBUF,)),
        ],
        compiler_params=pltpu.CompilerParams(needs_layout_passes=False),
    )
    def agg(x_hbm, packed_hbm, psum_a, psum_b, hist_hbm,
            slab_v, src_st, dst_st, rows2, hist_v, acc_sh, gsem):
        cid = lax.axis_index("c")
        sid = lax.axis_index("s")
        wid = cid * NS + sid
        rows_v = rows2.at[0]
        ones16 = jnp.ones((L,), jnp.float32)

        def unpack(c, b):
            def u(i, _):
                w = slab_v[pl.ds(c * CHUNK + i * L, L)]
                src_st[b, pl.ds(i * L, L)] = w & jnp.int32(0xFFFF)
                dst_st[b, pl.ds(i * L, L)] = lax.shift_right_logical(w, 16)
                return 0
            lax.fori_loop(0, CHUNK // L, u, 0)

        def zrow(i, _):
            def zcol(j, _):
                rows_v[i, pl.ds(j * L, L)] = jnp.zeros((L,), jnp.float32)
                return 0
            lax.fori_loop(0, D // L, zcol, 0)
            return 0
        lax.fori_loop(0, ZROWS, zrow, 0)

        def zhist(i, _):
            hist_v[pl.ds(i * L, L)] = jnp.zeros((L,), jnp.float32)
            return 0
        lax.fori_loop(0, HSIZE // L, zhist, 0)

        base = sid * ROWS_PER_TILE
        def zacc(t, _):
            pltpu.sync_copy(rows2.at[0, pl.ds(0, ZROWS)],
                            acc_sh.at[pl.ds(base + t * ZROWS, ZROWS)])
            return 0
        lax.fori_loop(0, ROWS_PER_TILE // ZROWS, zacc, 0)

        pltpu.sync_copy(packed_hbm.at[wid], slab_v)

        for b in range(NBUF):
            unpack(b, b)
            pltpu.async_copy(x_hbm.at[src_st.at[b]], rows2.at[b], gsem.at[b])

        plsc.subcore_barrier()

        scope = jax.named_scope("edge_loop")
        scope.__enter__()

        def pipe_body(t, _):
            j = t * NBUF
            for b in range(NBUF):
                cur = j + b
                def cnt(i, _):
                    idx16 = dst_st[b, pl.ds(i * L, L)]
                    plsc.addupdate_scatter(hist_v, [idx16], ones16)
                    return 0
                lax.fori_loop(0, CHUNK // L, cnt, 0)
                pltpu.make_async_copy(
                    x_hbm.at[src_st.at[b]], rows2.at[b], gsem.at[b]).wait()
                pltpu.sync_copy(rows2.at[b], acc_sh.at[dst_st.at[b]], add=True)
                nxt = cur + NBUF
                @pl.when(nxt < n_chunks)
                def _():
                    unpack(nxt, b)
                    pltpu.async_copy(
                        x_hbm.at[src_st.at[b]], rows2.at[b], gsem.at[b])
            return 0
        lax.fori_loop(0, n_chunks // NBUF, pipe_body, 0)
        scope.__exit__(None, None, None)

        pltpu.sync_copy(hist_v, hist_hbm.at[wid])
        plsc.subcore_barrier()

        sl = pl.ds(base, ROWS_PER_TILE)
        @pl.when(cid == 0)
        def _():
            pltpu.sync_copy(acc_sh.at[sl], psum_a.at[sl])
        @pl.when(cid == 1)
        def _():
            pltpu.sync_copy(acc_sh.at[sl], psum_b.at[sl])

    return agg(x, packed)


def _tc_self(x, W_self, b_self, b_neigh):
    n = x.shape[0]
    blk = 1024
    grid = (-(-n // blk),)

    def body(x_ref, ws_ref, bs_ref, bn_ref, o_ref):
        dn = (((1,), (1,)), ((), ()))
        o_ref[:] = (
            lax.dot_general(x_ref[:], ws_ref[:], dn,
                            preferred_element_type=jnp.float32)
            + bs_ref[:] + bn_ref[:]
        )

    return pl.pallas_call(
        body,
        grid=grid,
        in_specs=[
            pl.BlockSpec((blk, D), lambda i: (i, 0)),
            pl.BlockSpec((D, D), lambda i: (0, 0)),
            pl.BlockSpec((1, D), lambda i: (0, 0)),
            pl.BlockSpec((1, D), lambda i: (0, 0)),
        ],
        out_specs=pl.BlockSpec((blk, D), lambda i: (i, 0)),
        out_shape=jax.ShapeDtypeStruct((n, D), jnp.float32),
    )(x, W_self, b_self.reshape(1, D), b_neigh.reshape(1, D))


def _tc_combine(self_part, psum_a, psum_b, pcnt, W_neigh):
    n = self_part.shape[0]
    blk = 1024
    grid = (-(-n // blk),)

    def body(sp_ref, pa_ref, pb_ref, pc_ref, wn_ref, o_ref):
        s = pa_ref[:] + pb_ref[:]
        cnt = jnp.sum(pc_ref[:], axis=0)[:, None]
        mean = s / jnp.maximum(cnt, 1.0)
        dn = (((1,), (1,)), ((), ()))
        o_ref[:] = sp_ref[:] + lax.dot_general(
            mean, wn_ref[:], dn, preferred_element_type=jnp.float32)

    return pl.pallas_call(
        body,
        grid=grid,
        in_specs=[
            pl.BlockSpec((blk, D), lambda i: (i, 0)),
            pl.BlockSpec((blk, D), lambda i: (i, 0)),
            pl.BlockSpec((blk, D), lambda i: (i, 0)),
            pl.BlockSpec((NW, blk), lambda i: (0, i)),
            pl.BlockSpec((D, D), lambda i: (0, 0)),
        ],
        out_specs=pl.BlockSpec((blk, D), lambda i: (i, 0)),
        out_shape=jax.ShapeDtypeStruct((n, D), jnp.float32),
    )(self_part, psum_a, psum_b, pcnt, W_neigh)


def kernel(x, edge_index, W_self, b_self, W_neigh, b_neigh):
    n = x.shape[0]
    src = edge_index[0].astype(jnp.int32)
    dst = edge_index[1].astype(jnp.int32)
    e = src.shape[0]
    n_chunks = -(-e // (NW * CHUNK))
    n_chunks = max(-(-n_chunks // NBUF) * NBUF, NBUF)
    pad = NW * CHUNK * n_chunks - e
    pad_idx = jnp.arange(pad, dtype=jnp.int32)
    pad_src = jnp.minimum(pad_idx & 8191, n - 1)
    pad_dst = n + jnp.minimum(pad_idx & 127, HSIZE - n - 1)
    src_p = jnp.concatenate([src, pad_src])
    dst_p = jnp.concatenate([dst, pad_dst])
    packed = (src_p | (dst_p << 16)).reshape(NW, n_chunks * CHUNK)

    psum_a, psum_b, pcnt = _sc_aggregate(x, packed, n_chunks)

    self_part = _tc_self(x, W_self, b_self, b_neigh)
    return _tc_combine(self_part, psum_a, psum_b, pcnt, W_neigh)
